# baseline (speedup 1.0000x reference)
import functools

import jax
import jax.numpy as jnp
from jax import lax
from jax.experimental import pallas as pl
from jax.experimental.pallas import tpu as pltpu
from jax.experimental.pallas import tpu_sc as plsc

N_BLK = 2000
MP = 128
POS_THR = 0.15
NEG_THR = 0.7
BIGI = 2 ** 30

NW = 32
CHUNK = 624
L = 16


def _phase1_body(a_ref, gt_ref, rowmax_ref, t4v_ref, t4i_ref, vscr, iscr):
    j = pl.program_id(0)
    B = a_ref.shape[0]
    a = a_ref[...]
    ax1 = a[:, 0:1]
    ay1 = a[:, 1:2]
    ax2 = a[:, 2:3]
    ay2 = a[:, 3:4]
    ltx = jnp.maximum(ax1, gt_ref[0:1, :])
    lty = jnp.maximum(ay1, gt_ref[1:2, :])
    rbx = jnp.minimum(ax2, gt_ref[2:3, :])
    rby = jnp.minimum(ay2, gt_ref[3:4, :])
    whx = jnp.maximum(rbx - ltx, 0.0)
    why = jnp.maximum(rby - lty, 0.0)
    inter = whx * why
    areaa = (ax2 - ax1) * (ay2 - ay1)
    union = (areaa + gt_ref[4:5, :]) - inter
    iou = inter / jnp.maximum(union, 1e-6)

    rowmax_ref[...] = jnp.max(iou, axis=1, keepdims=True)

    @pl.when(j == 0)
    def _init():
        vscr[...] = jnp.full((8, MP), -1.0, jnp.float32)
        iscr[...] = jnp.full((8, MP), BIGI, jnp.int32)

    rowid = jax.lax.broadcasted_iota(jnp.int32, (B, MP), 0) + j * B
    cur = iou
    bv = []
    bi = []
    for t in range(4):
        cmax = jnp.max(cur, axis=0, keepdims=True)
        cand = jnp.where(cur == cmax, rowid, BIGI)
        cidx = jnp.min(cand, axis=0, keepdims=True)
        bv.append(cmax)
        bi.append(cidx)
        if t < 3:
            cur = jnp.where(rowid == cidx, -1.0, cur)
    comb_v = jnp.concatenate([vscr[0:4, :]] + bv, axis=0)
    comb_i = jnp.concatenate([iscr[0:4, :]] + bi, axis=0)
    nv = []
    ni = []
    for t in range(4):
        cmax = jnp.max(comb_v, axis=0, keepdims=True)
        cand = jnp.where(comb_v == cmax, comb_i, BIGI)
        cidx = jnp.min(cand, axis=0, keepdims=True)
        nv.append(cmax)
        ni.append(cidx)
        if t < 3:
            comb_v = jnp.where(comb_i == cidx, -2.0, comb_v)
    newv = jnp.concatenate(nv + nv, axis=0)
    newi = jnp.concatenate(ni + ni, axis=0)
    vscr[...] = newv
    iscr[...] = newi
    t4v_ref[...] = newv
    t4i_ref[...] = newi


def _sc_phase2(m_gt, rm_hbm, t4v_hbm, t4i_hbm, gtp_hbm, labi_hbm,
               lab_hbm, box_hbm,
               rm_v, t4v_v, t4i_v, gtp_v, labi_v,
               asg_v, lab_v, box_v):
    wid = lax.axis_index("s") * 2 + lax.axis_index("c")
    lane = jnp.arange(L, dtype=jnp.int32)
    lane_off = jnp.minimum(lane, 3) * MP
    lane_ok = lane < 4

    pltpu.sync_copy(t4v_hbm.at[pl.ds(0, 512)], t4v_v)
    pltpu.sync_copy(t4i_hbm.at[pl.ds(0, 512)], t4i_v)
    pltpu.sync_copy(gtp_hbm, gtp_v)
    pltpu.sync_copy(labi_hbm, labi_v)

    def process(base, nrows):
        ng = nrows // L
        pltpu.sync_copy(rm_hbm.at[pl.ds(base, nrows)], rm_v.at[pl.ds(0, nrows)])
        neg1 = jnp.full((L,), -1, jnp.int32)
        for g in range(ng):
            asg_v[pl.ds(g * L, L)] = neg1
        for m in range(m_gt):
            idxv = m + lane_off
            gidx = plsc.load_gather(t4i_v, [idxv])
            gval = plsc.load_gather(t4v_v, [idxv])
            valid = ((gval >= POS_THR) & lane_ok
                     & (gidx >= base) & (gidx < base + nrows))
            lidx = gidx - base
            val = jnp.full((L,), m + 1, jnp.int32)
            plsc.store_scatter(asg_v, [lidx], val, mask=valid)
        for g in range(ng):
            a = asg_v[pl.ds(g * L, L)]
            rmv = rm_v[pl.ds(g * L, L)]
            a2 = jnp.where(a == -1,
                           jnp.where(rmv < NEG_THR, 0, -1), a)
            pos = a2 > 0
            neg = a2 == 0
            safe = jnp.clip(a2 - 1, 0, m_gt - 1)
            labs = plsc.load_gather(labi_v, [safe])
            lab_v[pl.ds(g * L, L)] = jnp.where(
                pos, labs, jnp.where(neg, 0, -1))
            rows = g * L + lane
            for c in range(4):
                cc = plsc.load_gather(gtp_v, [safe, jnp.full((L,), c, jnp.int32)])
                vv = jnp.where(pos, cc, -1.0)
                plsc.store_scatter(box_v, [rows, jnp.full((L,), c, jnp.int32)], vv)
        pltpu.sync_copy(lab_v.at[pl.ds(0, nrows)], lab_hbm.at[pl.ds(base, nrows)])
        pltpu.sync_copy(box_v.at[pl.ds(0, nrows)], box_hbm.at[pl.ds(base, nrows)])

    process(wid * CHUNK, CHUNK)

    @pl.when(wid == NW - 1)
    def _tail():
        process(NW * CHUNK, 32)


def kernel(grid_bboxes, gt_bboxes, gt_labels):
    N = grid_bboxes.shape[0]
    M = gt_bboxes.shape[0]
    pad = jnp.full((MP - M, 4), -1e9, jnp.float32)
    gtp = jnp.concatenate([gt_bboxes, pad], axis=0)
    area_b = (gtp[:, 2] - gtp[:, 0]) * (gtp[:, 3] - gtp[:, 1])
    labp = jnp.concatenate(
        [gt_labels.astype(jnp.float32), jnp.zeros((MP - M,), jnp.float32)])
    zeros = jnp.zeros((MP,), jnp.float32)
    gtT = jnp.stack(
        [gtp[:, 0], gtp[:, 1], gtp[:, 2], gtp[:, 3], area_b, labp, zeros, zeros],
        axis=0)
    labi = jnp.concatenate(
        [gt_labels, jnp.zeros((MP - M,), jnp.int32)])
    nb = N // N_BLK

    rowmax, t4v, t4i = pl.pallas_call(
        _phase1_body,
        grid=(nb,),
        in_specs=[
            pl.BlockSpec((N_BLK, 4), lambda j: (j, 0)),
            pl.BlockSpec((8, MP), lambda j: (0, 0)),
        ],
        out_specs=[
            pl.BlockSpec((N_BLK, 1), lambda j: (j, 0)),
            pl.BlockSpec((8, MP), lambda j: (0, 0)),
            pl.BlockSpec((8, MP), lambda j: (0, 0)),
        ],
        out_shape=[
            jax.ShapeDtypeStruct((N, 1), jnp.float32),
            jax.ShapeDtypeStruct((8, MP), jnp.float32),
            jax.ShapeDtypeStruct((8, MP), jnp.int32),
        ],
        scratch_shapes=[
            pltpu.VMEM((8, MP), jnp.float32),
            pltpu.VMEM((8, MP), jnp.int32),
        ],
    )(grid_bboxes, gtT)

    mesh = plsc.VectorSubcoreMesh(core_axis_name="c", subcore_axis_name="s")
    sc = functools.partial(
        pl.kernel,
        mesh=mesh,
        compiler_params=pltpu.CompilerParams(needs_layout_passes=False),
        out_type=[
            jax.ShapeDtypeStruct((N,), jnp.int32),
            jax.ShapeDtypeStruct((N, 4), jnp.float32),
        ],
        scratch_types=[
            pltpu.VMEM((CHUNK,), jnp.float32),
            pltpu.VMEM((512,), jnp.float32),
            pltpu.VMEM((512,), jnp.int32),
            pltpu.VMEM((MP, 4), jnp.float32),
            pltpu.VMEM((MP,), jnp.int32),
            pltpu.VMEM((CHUNK,), jnp.int32),
            pltpu.VMEM((CHUNK,), jnp.int32),
            pltpu.VMEM((CHUNK, 4), jnp.float32),
        ],
    )(functools.partial(_sc_phase2, M))

    lab, boxes = sc(rowmax.reshape(N), t4v.reshape(8 * MP), t4i.reshape(8 * MP),
                    gtp, labi)
    return lab, boxes


# speedup vs baseline: 1.0273x; 1.0273x over previous
import functools

import jax
import jax.numpy as jnp
from jax import lax
from jax.experimental import pallas as pl
from jax.experimental.pallas import tpu as pltpu
from jax.experimental.pallas import tpu_sc as plsc

N_BLK = 2000
MP = 128
POS_THR = 0.15
NEG_THR = 0.7
BIGI = 2 ** 30

NW = 32
CHUNK = 624
L = 16


def _phase1_body(a_ref, gt_ref, rowmax_ref, t4v_ref, t4i_ref, vscr, iscr):
    j = pl.program_id(0)
    B = a_ref.shape[0]
    a = a_ref[...]
    ax1 = a[:, 0:1]
    ay1 = a[:, 1:2]
    ax2 = a[:, 2:3]
    ay2 = a[:, 3:4]
    ltx = jnp.maximum(ax1, gt_ref[0:1, :])
    lty = jnp.maximum(ay1, gt_ref[1:2, :])
    rbx = jnp.minimum(ax2, gt_ref[2:3, :])
    rby = jnp.minimum(ay2, gt_ref[3:4, :])
    whx = jnp.maximum(rbx - ltx, 0.0)
    why = jnp.maximum(rby - lty, 0.0)
    inter = whx * why
    areaa = (ax2 - ax1) * (ay2 - ay1)
    union = (areaa + gt_ref[4:5, :]) - inter
    iou = inter / jnp.maximum(union, 1e-6)

    rowmax_ref[...] = jnp.max(iou, axis=1, keepdims=True)

    @pl.when(j == 0)
    def _init():
        vscr[...] = jnp.full((8, MP), -1.0, jnp.float32)
        iscr[...] = jnp.full((8, MP), BIGI, jnp.int32)

    rowid = jax.lax.broadcasted_iota(jnp.int32, (B, MP), 0) + j * B
    cur = iou
    bv = []
    bi = []
    for t in range(4):
        cmax = jnp.max(cur, axis=0, keepdims=True)
        cand = jnp.where(cur == cmax, rowid, BIGI)
        cidx = jnp.min(cand, axis=0, keepdims=True)
        bv.append(cmax)
        bi.append(cidx)
        if t < 3:
            cur = jnp.where(rowid == cidx, -1.0, cur)
    comb_v = jnp.concatenate([vscr[0:4, :]] + bv, axis=0)
    comb_i = jnp.concatenate([iscr[0:4, :]] + bi, axis=0)
    nv = []
    ni = []
    for t in range(4):
        cmax = jnp.max(comb_v, axis=0, keepdims=True)
        cand = jnp.where(comb_v == cmax, comb_i, BIGI)
        cidx = jnp.min(cand, axis=0, keepdims=True)
        nv.append(cmax)
        ni.append(cidx)
        if t < 3:
            comb_v = jnp.where(comb_i == cidx, -2.0, comb_v)
    newv = jnp.concatenate(nv + nv, axis=0)
    newi = jnp.concatenate(ni + ni, axis=0)
    vscr[...] = newv
    iscr[...] = newi
    t4v_ref[...] = newv
    t4i_ref[...] = newi


def _sc_phase2(m_gt, rm_hbm, t4v_hbm, t4i_hbm, gtp_hbm, labi_hbm,
               lab_hbm, box_hbm,
               rm_v, t4v_v, t4i_v, gtp_v, labi_v,
               asg_v, lab_v, box_v,
               rm_t, asg_t, lab_t, box_t, dsem):
    wid = lax.axis_index("s") * 2 + lax.axis_index("c")
    base = wid * CHUNK
    lane = jnp.arange(L, dtype=jnp.int32)
    lane_off = jnp.minimum(lane, 3) * MP
    lane_ok = lane < 4
    neg1 = jnp.full((L,), -1, jnp.int32)

    d1 = pltpu.async_copy(t4v_hbm.at[pl.ds(0, 512)], t4v_v, dsem)
    d2 = pltpu.async_copy(t4i_hbm.at[pl.ds(0, 512)], t4i_v, dsem)
    d3 = pltpu.async_copy(gtp_hbm, gtp_v, dsem)
    d4 = pltpu.async_copy(labi_hbm, labi_v, dsem)
    d5 = pltpu.async_copy(rm_hbm.at[pl.ds(base, CHUNK)], rm_v, dsem)
    d6 = pltpu.async_copy(rm_hbm.at[pl.ds(NW * CHUNK, 32)], rm_t, dsem)
    for g in range(CHUNK // L):
        asg_v[pl.ds(g * L, L)] = neg1
    for g in range(32 // L):
        asg_t[pl.ds(g * L, L)] = neg1
    d1.wait()
    d2.wait()
    d3.wait()
    d4.wait()
    d5.wait()
    d6.wait()

    def process(asg, rm, lab, box, bs, nrows):
        for m in range(m_gt):
            idxv = m + lane_off
            gidx = plsc.load_gather(t4i_v, [idxv])
            gval = plsc.load_gather(t4v_v, [idxv])
            valid = ((gval >= POS_THR) & lane_ok
                     & (gidx >= bs) & (gidx < bs + nrows))
            lidx = gidx - bs
            val = jnp.full((L,), m + 1, jnp.int32)
            plsc.store_scatter(asg, [lidx], val, mask=valid)
        for g in range(nrows // L):
            a = asg[pl.ds(g * L, L)]
            rmv = rm[pl.ds(g * L, L)]
            a2 = jnp.where(a == -1,
                           jnp.where(rmv < NEG_THR, 0, -1), a)
            pos = a2 > 0
            neg = a2 == 0
            safe = jnp.clip(a2 - 1, 0, m_gt - 1)
            labs = plsc.load_gather(labi_v, [safe])
            lab[pl.ds(g * L, L)] = jnp.where(
                pos, labs, jnp.where(neg, 0, -1))
            rows = g * L + lane
            for c in range(4):
                cc = plsc.load_gather(gtp_v, [safe, jnp.full((L,), c, jnp.int32)])
                vv = jnp.where(pos, cc, -1.0)
                plsc.store_scatter(box, [rows, jnp.full((L,), c, jnp.int32)], vv)

    process(asg_v, rm_v, lab_v, box_v, base, CHUNK)
    o1 = pltpu.async_copy(lab_v, lab_hbm.at[pl.ds(base, CHUNK)], dsem)
    o2 = pltpu.async_copy(box_v, box_hbm.at[pl.ds(base, CHUNK)], dsem)

    @pl.when(wid == NW - 1)
    def _tail():
        process(asg_t, rm_t, lab_t, box_t, NW * CHUNK, 32)
        pltpu.sync_copy(lab_t, lab_hbm.at[pl.ds(NW * CHUNK, 32)])
        pltpu.sync_copy(box_t, box_hbm.at[pl.ds(NW * CHUNK, 32)])

    o1.wait()
    o2.wait()


def kernel(grid_bboxes, gt_bboxes, gt_labels):
    N = grid_bboxes.shape[0]
    M = gt_bboxes.shape[0]
    pad = jnp.full((MP - M, 4), -1e9, jnp.float32)
    gtp = jnp.concatenate([gt_bboxes, pad], axis=0)
    area_b = (gtp[:, 2] - gtp[:, 0]) * (gtp[:, 3] - gtp[:, 1])
    labp = jnp.concatenate(
        [gt_labels.astype(jnp.float32), jnp.zeros((MP - M,), jnp.float32)])
    zeros = jnp.zeros((MP,), jnp.float32)
    gtT = jnp.stack(
        [gtp[:, 0], gtp[:, 1], gtp[:, 2], gtp[:, 3], area_b, labp, zeros, zeros],
        axis=0)
    labi = jnp.concatenate(
        [gt_labels, jnp.zeros((MP - M,), jnp.int32)])
    nb = N // N_BLK

    rowmax, t4v, t4i = pl.pallas_call(
        _phase1_body,
        grid=(nb,),
        in_specs=[
            pl.BlockSpec((N_BLK, 4), lambda j: (j, 0)),
            pl.BlockSpec((8, MP), lambda j: (0, 0)),
        ],
        out_specs=[
            pl.BlockSpec((N_BLK, 1), lambda j: (j, 0)),
            pl.BlockSpec((8, MP), lambda j: (0, 0)),
            pl.BlockSpec((8, MP), lambda j: (0, 0)),
        ],
        out_shape=[
            jax.ShapeDtypeStruct((N, 1), jnp.float32),
            jax.ShapeDtypeStruct((8, MP), jnp.float32),
            jax.ShapeDtypeStruct((8, MP), jnp.int32),
        ],
        scratch_shapes=[
            pltpu.VMEM((8, MP), jnp.float32),
            pltpu.VMEM((8, MP), jnp.int32),
        ],
    )(grid_bboxes, gtT)

    mesh = plsc.VectorSubcoreMesh(core_axis_name="c", subcore_axis_name="s")
    sc = functools.partial(
        pl.kernel,
        mesh=mesh,
        compiler_params=pltpu.CompilerParams(needs_layout_passes=False),
        out_type=[
            jax.ShapeDtypeStruct((N,), jnp.int32),
            jax.ShapeDtypeStruct((N, 4), jnp.float32),
        ],
        scratch_types=[
            pltpu.VMEM((CHUNK,), jnp.float32),
            pltpu.VMEM((512,), jnp.float32),
            pltpu.VMEM((512,), jnp.int32),
            pltpu.VMEM((MP, 4), jnp.float32),
            pltpu.VMEM((MP,), jnp.int32),
            pltpu.VMEM((CHUNK,), jnp.int32),
            pltpu.VMEM((CHUNK,), jnp.int32),
            pltpu.VMEM((CHUNK, 4), jnp.float32),
            pltpu.VMEM((32,), jnp.float32),
            pltpu.VMEM((32,), jnp.int32),
            pltpu.VMEM((32,), jnp.int32),
            pltpu.VMEM((32, 4), jnp.float32),
            pltpu.SemaphoreType.DMA,
        ],
    )(functools.partial(_sc_phase2, M))

    lab, boxes = sc(rowmax.reshape(N), t4v.reshape(8 * MP), t4i.reshape(8 * MP),
                    gtp, labi)
    return lab, boxes
